# baseline (speedup 1.0000x reference)
import jax
import jax.numpy as jnp
from jax.experimental import pallas as pl
from jax.experimental.pallas import tpu as pltpu

_TILE_C = 256


def _fused_mlp_kernel(x_ref, w1_ref, b1_ref, w2_ref, b2_ref, w3_ref, b3_ref,
                      o_ref):
    x = x_ref[0]
    dot = lambda w, h: jax.lax.dot_general(
        w, h, (((1,), (0,)), ((), ())), preferred_element_type=jnp.float32)
    h = jnp.maximum(dot(w1_ref[...], x) + b1_ref[...], 0.0)
    h = jnp.maximum(dot(w2_ref[...], h) + b2_ref[...], 0.0)
    o_ref[0] = jnp.maximum(dot(w3_ref[...], h) + b3_ref[...], 0.0)


def kernel(x, scale0_params, scale1_params, scale2_params, fusion_params):
    del scale0_params, scale1_params, scale2_params
    (w1, b1), (w2, b2), (w3, b3) = fusion_params
    B, C_in, N = x.shape
    C_out, C_mid = w3.shape
    tile_c = min(_TILE_C, C_out)
    grid = (B, C_out // tile_c)

    full = lambda shape: pl.BlockSpec(shape, lambda b, c: (0,) * len(shape))
    return pl.pallas_call(
        _fused_mlp_kernel,
        grid=grid,
        in_specs=[
            pl.BlockSpec((1, C_in, N), lambda b, c: (b, 0, 0)),
            full(w1.shape), full((w1.shape[0], 1)),
            full(w2.shape), full((w2.shape[0], 1)),
            pl.BlockSpec((tile_c, C_mid), lambda b, c: (c, 0)),
            pl.BlockSpec((tile_c, 1), lambda b, c: (c, 0)),
        ],
        out_specs=pl.BlockSpec((1, tile_c, N), lambda b, c: (b, c, 0)),
        out_shape=jax.ShapeDtypeStruct((B, C_out, N), jnp.float32),
        compiler_params=pltpu.CompilerParams(
            dimension_semantics=("parallel", "parallel")),
    )(x, w1, b1[:, None], w2, b2[:, None], w3, b3[:, None])


# speedup vs baseline: 1.0319x; 1.0319x over previous
import jax
import jax.numpy as jnp
from jax.experimental import pallas as pl
from jax.experimental.pallas import tpu as pltpu

_TILE_C = 256


def _fused_mlp_kernel(x_ref, w1_ref, b1_ref, w2_ref, b2_ref, w3_ref, b3_ref,
                      o_ref, h_ref):
    dot = lambda w, h: jax.lax.dot_general(
        w, h, (((1,), (0,)), ((), ())), preferred_element_type=jnp.float32)

    @pl.when(pl.program_id(1) == 0)
    def _():
        h1 = jnp.maximum(dot(w1_ref[...], x_ref[0]) + b1_ref[...], 0.0)
        h_ref[...] = jnp.maximum(dot(w2_ref[...], h1) + b2_ref[...], 0.0)

    o_ref[0] = jnp.maximum(dot(w3_ref[...], h_ref[...]) + b3_ref[...], 0.0)


def kernel(x, scale0_params, scale1_params, scale2_params, fusion_params):
    del scale0_params, scale1_params, scale2_params
    (w1, b1), (w2, b2), (w3, b3) = fusion_params
    B, C_in, N = x.shape
    C_out, C_mid = w3.shape
    tile_c = min(_TILE_C, C_out)
    grid = (B, C_out // tile_c)

    full = lambda shape: pl.BlockSpec(shape, lambda b, c: (0,) * len(shape))
    return pl.pallas_call(
        _fused_mlp_kernel,
        grid=grid,
        in_specs=[
            pl.BlockSpec((1, C_in, N), lambda b, c: (b, 0, 0)),
            full(w1.shape), full((w1.shape[0], 1)),
            full(w2.shape), full((w2.shape[0], 1)),
            pl.BlockSpec((tile_c, C_mid), lambda b, c: (c, 0)),
            pl.BlockSpec((tile_c, 1), lambda b, c: (c, 0)),
        ],
        out_specs=pl.BlockSpec((1, tile_c, N), lambda b, c: (b, c, 0)),
        out_shape=jax.ShapeDtypeStruct((B, C_out, N), jnp.float32),
        scratch_shapes=[pltpu.VMEM((C_mid, N), jnp.float32)],
        compiler_params=pltpu.CompilerParams(
            dimension_semantics=("arbitrary", "arbitrary")),
    )(x, w1, b1[:, None], w2, b2[:, None], w3, b3[:, None])


# speedup vs baseline: 1.4081x; 1.3646x over previous
import jax
import jax.numpy as jnp
from jax.experimental import pallas as pl
from jax.experimental.pallas import tpu as pltpu

_TILE_B = 2


def _fused_mlp_kernel(x_ref, w1_ref, b1_ref, w2_ref, b2_ref, w3_ref, b3_ref,
                      o_ref):
    dot = lambda w, h: jax.lax.dot_general(
        w, h, (((1,), (0,)), ((), ())), preferred_element_type=jnp.float32)
    for i in range(x_ref.shape[0]):
        h = jnp.maximum(dot(w1_ref[...], x_ref[i]) + b1_ref[...], 0.0)
        h = jnp.maximum(dot(w2_ref[...], h) + b2_ref[...], 0.0)
        o_ref[i] = jnp.maximum(dot(w3_ref[...], h) + b3_ref[...], 0.0)


def kernel(x, scale0_params, scale1_params, scale2_params, fusion_params):
    del scale0_params, scale1_params, scale2_params
    (w1, b1), (w2, b2), (w3, b3) = fusion_params
    B, C_in, N = x.shape
    C_out = w3.shape[0]
    tile_b = min(_TILE_B, B)
    grid = (B // tile_b,)

    full = lambda shape: pl.BlockSpec(shape, lambda b: (0,) * len(shape))
    return pl.pallas_call(
        _fused_mlp_kernel,
        grid=grid,
        in_specs=[
            pl.BlockSpec((tile_b, C_in, N), lambda b: (b, 0, 0)),
            full(w1.shape), full((w1.shape[0], 1)),
            full(w2.shape), full((w2.shape[0], 1)),
            full(w3.shape), full((w3.shape[0], 1)),
        ],
        out_specs=pl.BlockSpec((tile_b, C_out, N), lambda b: (b, 0, 0)),
        out_shape=jax.ShapeDtypeStruct((B, C_out, N), jnp.float32),
        compiler_params=pltpu.CompilerParams(
            dimension_semantics=("parallel",)),
    )(x, w1, b1[:, None], w2, b2[:, None], w3, b3[:, None])


# speedup vs baseline: 1.4861x; 1.0554x over previous
import jax
import jax.numpy as jnp
from jax.experimental import pallas as pl
from jax.experimental.pallas import tpu as pltpu

_TILE_B = 1


def _fused_mlp_kernel(x_ref, w1_ref, b1_ref, w2_ref, b2_ref, w3_ref, b3_ref,
                      o_ref):
    dot = lambda w, h: jax.lax.dot_general(
        w, h, (((1,), (0,)), ((), ())), preferred_element_type=jnp.float32)
    for i in range(x_ref.shape[0]):
        h = jnp.maximum(dot(w1_ref[...], x_ref[i]) + b1_ref[...], 0.0)
        h = jnp.maximum(dot(w2_ref[...], h) + b2_ref[...], 0.0)
        o_ref[i] = jnp.maximum(dot(w3_ref[...], h) + b3_ref[...], 0.0)


def kernel(x, scale0_params, scale1_params, scale2_params, fusion_params):
    del scale0_params, scale1_params, scale2_params
    (w1, b1), (w2, b2), (w3, b3) = fusion_params
    B, C_in, N = x.shape
    C_out = w3.shape[0]
    tile_b = min(_TILE_B, B)
    grid = (B // tile_b,)

    full = lambda shape: pl.BlockSpec(shape, lambda b: (0,) * len(shape))
    return pl.pallas_call(
        _fused_mlp_kernel,
        grid=grid,
        in_specs=[
            pl.BlockSpec((tile_b, C_in, N), lambda b: (b, 0, 0)),
            full(w1.shape), full((w1.shape[0], 1)),
            full(w2.shape), full((w2.shape[0], 1)),
            full(w3.shape), full((w3.shape[0], 1)),
        ],
        out_specs=pl.BlockSpec((tile_b, C_out, N), lambda b: (b, 0, 0)),
        out_shape=jax.ShapeDtypeStruct((B, C_out, N), jnp.float32),
        compiler_params=pltpu.CompilerParams(
            dimension_semantics=("parallel",)),
    )(x, w1, b1[:, None], w2, b2[:, None], w3, b3[:, None])
